# baseline (speedup 1.0000x reference)
import functools

import jax
import jax.numpy as jnp
from jax import lax
from jax.experimental import pallas as pl
from jax.experimental.pallas import tpu as pltpu
from jax.experimental.pallas import tpu_sc as plsc

N_NODES = 10000
N_EDGES = 320000
IN_F = 128
HID = 256
OUT_F = 32

N_SUB = 16
K = 100
JS = N_EDGES // N_SUB // K
N_PAD = 10240
ROWS_PER_S = N_PAD // N_SUB
DEG_W = 8

_mesh = plsc.VectorSubcoreMesh(core_axis_name="c", subcore_axis_name="s")
_sc_params = pltpu.CompilerParams(use_tc_tiling_on_sc=False)


@functools.partial(
    pl.kernel,
    mesh=_mesh,
    out_type=jax.ShapeDtypeStruct((2, N_PAD, DEG_W), jnp.float32),
    scratch_types=[
        pltpu.VMEM((JS, K), jnp.int32),
        pltpu.VMEM((K, DEG_W), jnp.float32),
        pltpu.VMEM_SHARED((N_PAD, DEG_W), jnp.float32),
        pltpu.SemaphoreType.DMA((10,)),
    ],
    compiler_params=_sc_params,
)
def _deg_kernel(edge_hbm, ones_hbm, zeros_hbm, out_hbm, idx_v, ones_v, acc,
                dsem):
    cid = lax.axis_index("c")
    sid = lax.axis_index("s")
    zrow = pl.ds(sid * ROWS_PER_S, ROWS_PER_S)
    pltpu.sync_copy(zeros_hbm, acc.at[zrow])
    pltpu.sync_copy(ones_hbm, ones_v)
    pltpu.sync_copy(edge_hbm.at[cid].at[sid], idx_v)
    plsc.subcore_barrier()

    def group(g, carry):
        base = g * 10
        for u in range(10):
            pltpu.async_copy(ones_v, acc.at[idx_v.at[base + u]],
                             dsem.at[u], add=True)
        for u in range(10):
            pltpu.make_async_copy(ones_v, acc.at[idx_v.at[base]],
                                  dsem.at[u]).wait()
        return carry

    lax.fori_loop(0, JS // 10, group, 0)
    plsc.subcore_barrier()
    pltpu.sync_copy(acc.at[zrow], out_hbm.at[cid].at[zrow])


def _make_agg(width, nbuf, split_edges):
    js = JS // 2 if split_edges else JS

    @functools.partial(
        pl.kernel,
        mesh=_mesh,
        out_type=jax.ShapeDtypeStruct((2, N_PAD, width), jnp.float32),
        scratch_types=[
            pltpu.VMEM((js, K), jnp.int32),
            pltpu.VMEM((js, K), jnp.int32),
            pltpu.VMEM((nbuf, K, width), jnp.float32),
            pltpu.VMEM_SHARED((N_PAD, width), jnp.float32),
            pltpu.SemaphoreType.DMA((nbuf,)),
            pltpu.SemaphoreType.DMA((nbuf,)),
        ],
        compiler_params=_sc_params,
    )
    def agg(*refs):
        nh = 1 if split_edges else 2
        h_hbms = refs[:nh]
        edge_hbm, zeros_hbm, out_hbm = refs[nh:nh + 3]
        src_v, dst_v, rows_v, acc, gsem, ssem = refs[nh + 3:]
        cid = lax.axis_index("c")
        sid = lax.axis_index("s")
        zrow = pl.ds(sid * ROWS_PER_S, ROWS_PER_S)

        pltpu.sync_copy(zeros_hbm, acc.at[zrow])
        if split_edges:
            rows = pl.ds(cid * js, js)
            pltpu.sync_copy(edge_hbm.at[0].at[sid].at[rows], src_v)
            pltpu.sync_copy(edge_hbm.at[1].at[sid].at[rows], dst_v)
        else:
            pltpu.sync_copy(edge_hbm.at[0].at[sid], src_v)
            pltpu.sync_copy(edge_hbm.at[1].at[sid], dst_v)
        plsc.subcore_barrier()

        ngroups = js // nbuf
        tail = js - ngroups * nbuf

        def run(h_hbm):
            def group(g, carry):
                base = g * nbuf
                for b in range(nbuf):
                    @pl.when(g > 0)
                    def _():
                        pltpu.make_async_copy(
                            rows_v.at[b], acc.at[dst_v.at[base + b]],
                            ssem.at[b]).wait()
                    pltpu.async_copy(h_hbm.at[src_v.at[base + b]],
                                     rows_v.at[b], gsem.at[b])
                for b in range(nbuf):
                    pltpu.make_async_copy(h_hbm.at[src_v.at[base + b]],
                                          rows_v.at[b], gsem.at[b]).wait()
                    pltpu.async_copy(rows_v.at[b], acc.at[dst_v.at[base + b]],
                                     ssem.at[b], add=True)
                return carry

            lax.fori_loop(0, ngroups, group, 0)
            for u in range(tail):
                jt = ngroups * nbuf + u
                pltpu.make_async_copy(rows_v.at[u], acc.at[dst_v.at[jt]],
                                      ssem.at[u]).wait()
                pltpu.async_copy(h_hbm.at[src_v.at[jt]],
                                 rows_v.at[u], gsem.at[u]).wait()
                pltpu.async_copy(rows_v.at[u], acc.at[dst_v.at[jt]],
                                 ssem.at[u], add=True)
            for b in range(nbuf):
                pltpu.make_async_copy(rows_v.at[b], acc.at[dst_v.at[b]],
                                      ssem.at[b]).wait()

        if split_edges:
            run(h_hbms[0])
        else:
            @pl.when(cid == 0)
            def _():
                run(h_hbms[0])

            @pl.when(cid == 1)
            def _():
                run(h_hbms[1])

        plsc.subcore_barrier()
        pltpu.sync_copy(acc.at[zrow], out_hbm.at[cid].at[zrow])

    return agg


_agg64 = _make_agg(IN_F // 2, 5, split_edges=False)
_agg32 = _make_agg(OUT_F, 5, split_edges=True)


_R = 2000


def _pre_body(x_ref, deg_ref, h0_ref, h1_ref, ns_ref, nd_ref):
    ns = lax.rsqrt(jnp.maximum(deg_ref[0, :, :1], 1.0))
    nd = lax.rsqrt(jnp.maximum(deg_ref[1, :, :1], 1.0))
    h = x_ref[...] * ns
    h0_ref[...] = h[:, :IN_F // 2]
    h1_ref[...] = h[:, IN_F // 2:]
    ns_ref[...] = ns
    nd_ref[...] = nd


_pre = pl.pallas_call(
    _pre_body,
    grid=(N_NODES // _R,),
    in_specs=[
        pl.BlockSpec((_R, IN_F), lambda i: (i, 0)),
        pl.BlockSpec((2, _R, DEG_W), lambda i: (0, i, 0)),
    ],
    out_specs=[
        pl.BlockSpec((_R, IN_F // 2), lambda i: (i, 0)),
        pl.BlockSpec((_R, IN_F // 2), lambda i: (i, 0)),
        pl.BlockSpec((_R, 1), lambda i: (i, 0)),
        pl.BlockSpec((_R, 1), lambda i: (i, 0)),
    ],
    out_shape=[
        jax.ShapeDtypeStruct((N_NODES, IN_F // 2), jnp.float32),
        jax.ShapeDtypeStruct((N_NODES, IN_F // 2), jnp.float32),
        jax.ShapeDtypeStruct((N_NODES, 1), jnp.float32),
        jax.ShapeDtypeStruct((N_NODES, 1), jnp.float32),
    ],
)


def _mid_body(p_ref, nd_ref, ns_ref, w1_ref, b1_ref, w2_ref, t_ref):
    a0 = p_ref[0] * nd_ref[...]
    a1 = p_ref[1] * nd_ref[...]
    z = (jnp.dot(a0, w1_ref[:IN_F // 2], preferred_element_type=jnp.float32)
         + jnp.dot(a1, w1_ref[IN_F // 2:], preferred_element_type=jnp.float32)
         + b1_ref[...])
    r = jnp.maximum(z, 0.0)
    t_ref[...] = jnp.dot(r, w2_ref[...],
                         preferred_element_type=jnp.float32) * ns_ref[...]


_mid = pl.pallas_call(
    _mid_body,
    grid=(N_NODES // _R,),
    in_specs=[
        pl.BlockSpec((2, _R, IN_F // 2), lambda i: (0, i, 0)),
        pl.BlockSpec((_R, 1), lambda i: (i, 0)),
        pl.BlockSpec((_R, 1), lambda i: (i, 0)),
        pl.BlockSpec((IN_F, HID), lambda i: (0, 0)),
        pl.BlockSpec((1, HID), lambda i: (0, 0)),
        pl.BlockSpec((HID, OUT_F), lambda i: (0, 0)),
    ],
    out_specs=pl.BlockSpec((_R, OUT_F), lambda i: (i, 0)),
    out_shape=jax.ShapeDtypeStruct((N_NODES, OUT_F), jnp.float32),
)


def _out_body(q_ref, nd_ref, b2_ref, o_ref):
    o_ref[...] = (q_ref[0] + q_ref[1]) * nd_ref[...] + b2_ref[...]


_out = pl.pallas_call(
    _out_body,
    grid=(N_NODES // _R,),
    in_specs=[
        pl.BlockSpec((2, _R, OUT_F), lambda i: (0, i, 0)),
        pl.BlockSpec((_R, 1), lambda i: (i, 0)),
        pl.BlockSpec((1, OUT_F), lambda i: (0, 0)),
    ],
    out_specs=pl.BlockSpec((_R, OUT_F), lambda i: (i, 0)),
    out_shape=jax.ShapeDtypeStruct((N_NODES, OUT_F), jnp.float32),
)


def kernel(features, edge_index, W1, b1, W2, b2):
    edge4d = edge_index.astype(jnp.int32).reshape(2, N_SUB, JS, K)
    ones8 = jnp.ones((K, DEG_W), jnp.float32)
    zeros8 = jnp.zeros((ROWS_PER_S, DEG_W), jnp.float32)
    deg = _deg_kernel(edge4d, ones8, zeros8)
    h0, h1, ns, nd = _pre(features, deg)
    zeros64 = jnp.zeros((ROWS_PER_S, IN_F // 2), jnp.float32)
    p = _agg64(h0, h1, edge4d, zeros64)
    t = _mid(p, nd, ns, W1, b1.reshape(1, HID), W2)
    zeros32 = jnp.zeros((ROWS_PER_S, OUT_F), jnp.float32)
    q = _agg32(t, edge4d, zeros32)
    return _out(q, nd, b2.reshape(1, OUT_F))


# speedup vs baseline: 1.0510x; 1.0510x over previous
import functools

import jax
import jax.numpy as jnp
from jax import lax
from jax.experimental import pallas as pl
from jax.experimental.pallas import tpu as pltpu
from jax.experimental.pallas import tpu_sc as plsc

N_NODES = 10000
N_EDGES = 320000
IN_F = 128
HID = 256
OUT_F = 32

N_SUB = 16
K = 80
JS = N_EDGES // N_SUB // K
N_PAD = 10240
ROWS_PER_S = N_PAD // N_SUB
DEG_W = 8

_mesh = plsc.VectorSubcoreMesh(core_axis_name="c", subcore_axis_name="s")
_sc_params = pltpu.CompilerParams(use_tc_tiling_on_sc=False)


@functools.partial(
    pl.kernel,
    mesh=_mesh,
    out_type=jax.ShapeDtypeStruct((2, N_PAD, DEG_W), jnp.float32),
    scratch_types=[
        pltpu.VMEM((JS, K), jnp.int32),
        pltpu.VMEM((K, DEG_W), jnp.float32),
        pltpu.VMEM_SHARED((N_PAD, DEG_W), jnp.float32),
        pltpu.SemaphoreType.DMA((10,)),
    ],
    compiler_params=_sc_params,
)
def _deg_kernel(edge_hbm, ones_hbm, zeros_hbm, out_hbm, idx_v, ones_v, acc,
                dsem):
    cid = lax.axis_index("c")
    sid = lax.axis_index("s")
    zrow = pl.ds(sid * ROWS_PER_S, ROWS_PER_S)
    pltpu.sync_copy(zeros_hbm, acc.at[zrow])
    pltpu.sync_copy(ones_hbm, ones_v)
    pltpu.sync_copy(edge_hbm.at[cid].at[sid], idx_v)
    plsc.subcore_barrier()

    def group(g, carry):
        base = g * 10
        for u in range(10):
            pltpu.async_copy(ones_v, acc.at[idx_v.at[base + u]],
                             dsem.at[u], add=True)
        for u in range(10):
            pltpu.make_async_copy(ones_v, acc.at[idx_v.at[base]],
                                  dsem.at[u]).wait()
        return carry

    lax.fori_loop(0, JS // 10, group, 0)
    plsc.subcore_barrier()
    pltpu.sync_copy(acc.at[zrow], out_hbm.at[cid].at[zrow])


def _make_agg(width, nbuf, split_edges):
    js = JS // 2 if split_edges else JS

    @functools.partial(
        pl.kernel,
        mesh=_mesh,
        out_type=jax.ShapeDtypeStruct((2, N_PAD, width), jnp.float32),
        scratch_types=[
            pltpu.VMEM((js, K), jnp.int32),
            pltpu.VMEM((js, K), jnp.int32),
            pltpu.VMEM((nbuf, K, width), jnp.float32),
            pltpu.VMEM_SHARED((N_PAD, width), jnp.float32),
            pltpu.SemaphoreType.DMA((nbuf,)),
            pltpu.SemaphoreType.DMA((nbuf,)),
        ],
        compiler_params=_sc_params,
    )
    def agg(*refs):
        nh = 1 if split_edges else 2
        h_hbms = refs[:nh]
        edge_hbm, zeros_hbm, out_hbm = refs[nh:nh + 3]
        src_v, dst_v, rows_v, acc, gsem, ssem = refs[nh + 3:]
        cid = lax.axis_index("c")
        sid = lax.axis_index("s")
        zrow = pl.ds(sid * ROWS_PER_S, ROWS_PER_S)

        pltpu.sync_copy(zeros_hbm, acc.at[zrow])
        if split_edges:
            rows = pl.ds(cid * js, js)
            pltpu.sync_copy(edge_hbm.at[0].at[sid].at[rows], src_v)
            pltpu.sync_copy(edge_hbm.at[1].at[sid].at[rows], dst_v)
        else:
            pltpu.sync_copy(edge_hbm.at[0].at[sid], src_v)
            pltpu.sync_copy(edge_hbm.at[1].at[sid], dst_v)
        plsc.subcore_barrier()

        ngroups = js // nbuf
        tail = js - ngroups * nbuf

        def run(h_hbm):
            def group(g, carry):
                base = g * nbuf
                for b in range(nbuf):
                    @pl.when(g > 0)
                    def _():
                        pltpu.make_async_copy(
                            rows_v.at[b], acc.at[dst_v.at[base + b]],
                            ssem.at[b]).wait()
                    pltpu.async_copy(h_hbm.at[src_v.at[base + b]],
                                     rows_v.at[b], gsem.at[b])
                for b in range(nbuf):
                    pltpu.make_async_copy(h_hbm.at[src_v.at[base + b]],
                                          rows_v.at[b], gsem.at[b]).wait()
                    pltpu.async_copy(rows_v.at[b], acc.at[dst_v.at[base + b]],
                                     ssem.at[b], add=True)
                return carry

            lax.fori_loop(0, ngroups, group, 0)
            for u in range(tail):
                jt = ngroups * nbuf + u
                pltpu.make_async_copy(rows_v.at[u], acc.at[dst_v.at[jt]],
                                      ssem.at[u]).wait()
                pltpu.async_copy(h_hbm.at[src_v.at[jt]],
                                 rows_v.at[u], gsem.at[u]).wait()
                pltpu.async_copy(rows_v.at[u], acc.at[dst_v.at[jt]],
                                 ssem.at[u], add=True)
            for b in range(nbuf):
                pltpu.make_async_copy(rows_v.at[b], acc.at[dst_v.at[b]],
                                      ssem.at[b]).wait()

        if split_edges:
            run(h_hbms[0])
        else:
            @pl.when(cid == 0)
            def _():
                run(h_hbms[0])

            @pl.when(cid == 1)
            def _():
                run(h_hbms[1])

        plsc.subcore_barrier()
        pltpu.sync_copy(acc.at[zrow], out_hbm.at[cid].at[zrow])

    return agg


_agg64 = _make_agg(IN_F // 2, 8, split_edges=False)
_agg32 = _make_agg(OUT_F, 5, split_edges=True)


_R = 2000


def _pre_body(x_ref, deg_ref, h0_ref, h1_ref, ns_ref, nd_ref):
    ns = lax.rsqrt(jnp.maximum(deg_ref[0, :, :1], 1.0))
    nd = lax.rsqrt(jnp.maximum(deg_ref[1, :, :1], 1.0))
    h = x_ref[...] * ns
    h0_ref[...] = h[:, :IN_F // 2]
    h1_ref[...] = h[:, IN_F // 2:]
    ns_ref[...] = ns
    nd_ref[...] = nd


_pre = pl.pallas_call(
    _pre_body,
    grid=(N_NODES // _R,),
    in_specs=[
        pl.BlockSpec((_R, IN_F), lambda i: (i, 0)),
        pl.BlockSpec((2, _R, DEG_W), lambda i: (0, i, 0)),
    ],
    out_specs=[
        pl.BlockSpec((_R, IN_F // 2), lambda i: (i, 0)),
        pl.BlockSpec((_R, IN_F // 2), lambda i: (i, 0)),
        pl.BlockSpec((_R, 1), lambda i: (i, 0)),
        pl.BlockSpec((_R, 1), lambda i: (i, 0)),
    ],
    out_shape=[
        jax.ShapeDtypeStruct((N_NODES, IN_F // 2), jnp.float32),
        jax.ShapeDtypeStruct((N_NODES, IN_F // 2), jnp.float32),
        jax.ShapeDtypeStruct((N_NODES, 1), jnp.float32),
        jax.ShapeDtypeStruct((N_NODES, 1), jnp.float32),
    ],
)


def _mid_body(p_ref, nd_ref, ns_ref, w1_ref, b1_ref, w2_ref, t_ref):
    a0 = p_ref[0] * nd_ref[...]
    a1 = p_ref[1] * nd_ref[...]
    z = (jnp.dot(a0, w1_ref[:IN_F // 2], preferred_element_type=jnp.float32)
         + jnp.dot(a1, w1_ref[IN_F // 2:], preferred_element_type=jnp.float32)
         + b1_ref[...])
    r = jnp.maximum(z, 0.0)
    t_ref[...] = jnp.dot(r, w2_ref[...],
                         preferred_element_type=jnp.float32) * ns_ref[...]


_mid = pl.pallas_call(
    _mid_body,
    grid=(N_NODES // _R,),
    in_specs=[
        pl.BlockSpec((2, _R, IN_F // 2), lambda i: (0, i, 0)),
        pl.BlockSpec((_R, 1), lambda i: (i, 0)),
        pl.BlockSpec((_R, 1), lambda i: (i, 0)),
        pl.BlockSpec((IN_F, HID), lambda i: (0, 0)),
        pl.BlockSpec((1, HID), lambda i: (0, 0)),
        pl.BlockSpec((HID, OUT_F), lambda i: (0, 0)),
    ],
    out_specs=pl.BlockSpec((_R, OUT_F), lambda i: (i, 0)),
    out_shape=jax.ShapeDtypeStruct((N_NODES, OUT_F), jnp.float32),
)


def _out_body(q_ref, nd_ref, b2_ref, o_ref):
    o_ref[...] = (q_ref[0] + q_ref[1]) * nd_ref[...] + b2_ref[...]


_out = pl.pallas_call(
    _out_body,
    grid=(N_NODES // _R,),
    in_specs=[
        pl.BlockSpec((2, _R, OUT_F), lambda i: (0, i, 0)),
        pl.BlockSpec((_R, 1), lambda i: (i, 0)),
        pl.BlockSpec((1, OUT_F), lambda i: (0, 0)),
    ],
    out_specs=pl.BlockSpec((_R, OUT_F), lambda i: (i, 0)),
    out_shape=jax.ShapeDtypeStruct((N_NODES, OUT_F), jnp.float32),
)


def kernel(features, edge_index, W1, b1, W2, b2):
    edge4d = edge_index.astype(jnp.int32).reshape(2, N_SUB, JS, K)
    ones8 = jnp.ones((K, DEG_W), jnp.float32)
    zeros8 = jnp.zeros((ROWS_PER_S, DEG_W), jnp.float32)
    deg = _deg_kernel(edge4d, ones8, zeros8)
    h0, h1, ns, nd = _pre(features, deg)
    zeros64 = jnp.zeros((ROWS_PER_S, IN_F // 2), jnp.float32)
    p = _agg64(h0, h1, edge4d, zeros64)
    t = _mid(p, nd, ns, W1, b1.reshape(1, HID), W2)
    zeros32 = jnp.zeros((ROWS_PER_S, OUT_F), jnp.float32)
    q = _agg32(t, edge4d, zeros32)
    return _out(q, nd, b2.reshape(1, OUT_F))


# speedup vs baseline: 1.0517x; 1.0006x over previous
import functools

import jax
import jax.numpy as jnp
from jax import lax
from jax.experimental import pallas as pl
from jax.experimental.pallas import tpu as pltpu
from jax.experimental.pallas import tpu_sc as plsc

N_NODES = 10000
N_EDGES = 320000
IN_F = 128
HID = 256
OUT_F = 32

N_SUB = 16
K = 80
JS = N_EDGES // N_SUB // K
N_PAD = 10240
ROWS_PER_S = N_PAD // N_SUB
DEG_W = 8

_mesh = plsc.VectorSubcoreMesh(core_axis_name="c", subcore_axis_name="s")
_sc_params = pltpu.CompilerParams(use_tc_tiling_on_sc=False)


@functools.partial(
    pl.kernel,
    mesh=_mesh,
    out_type=jax.ShapeDtypeStruct((2, N_PAD, DEG_W), jnp.float32),
    scratch_types=[
        pltpu.VMEM((JS, K), jnp.int32),
        pltpu.VMEM((K, DEG_W), jnp.float32),
        pltpu.VMEM_SHARED((N_PAD, DEG_W), jnp.float32),
        pltpu.SemaphoreType.DMA((10,)),
    ],
    compiler_params=_sc_params,
)
def _deg_kernel(edge_hbm, ones_hbm, zeros_hbm, out_hbm, idx_v, ones_v, acc,
                dsem):
    cid = lax.axis_index("c")
    sid = lax.axis_index("s")
    zrow = pl.ds(sid * ROWS_PER_S, ROWS_PER_S)
    pltpu.sync_copy(zeros_hbm, acc.at[zrow])
    pltpu.sync_copy(ones_hbm, ones_v)
    pltpu.sync_copy(edge_hbm.at[cid].at[sid], idx_v)
    plsc.subcore_barrier()

    def group(g, carry):
        base = g * 10
        for u in range(10):
            pltpu.async_copy(ones_v, acc.at[idx_v.at[base + u]],
                             dsem.at[u], add=True)
        for u in range(10):
            pltpu.make_async_copy(ones_v, acc.at[idx_v.at[base]],
                                  dsem.at[u]).wait()
        return carry

    lax.fori_loop(0, JS // 10, group, 0)
    plsc.subcore_barrier()
    pltpu.sync_copy(acc.at[zrow], out_hbm.at[cid].at[zrow])


def _make_agg(width, nbuf, split_edges):
    js = JS // 2 if split_edges else JS

    @functools.partial(
        pl.kernel,
        mesh=_mesh,
        out_type=jax.ShapeDtypeStruct((2, N_PAD, width), jnp.float32),
        scratch_types=[
            pltpu.VMEM((js, K), jnp.int32),
            pltpu.VMEM((js, K), jnp.int32),
            pltpu.VMEM((nbuf, K, width), jnp.float32),
            pltpu.VMEM_SHARED((N_PAD, width), jnp.float32),
            pltpu.SemaphoreType.DMA((nbuf,)),
            pltpu.SemaphoreType.DMA((nbuf,)),
        ],
        compiler_params=_sc_params,
    )
    def agg(*refs):
        nh = 1 if split_edges else 2
        h_hbms = refs[:nh]
        edge_hbm, zeros_hbm, out_hbm = refs[nh:nh + 3]
        src_v, dst_v, rows_v, acc, gsem, ssem = refs[nh + 3:]
        cid = lax.axis_index("c")
        sid = lax.axis_index("s")
        zrow = pl.ds(sid * ROWS_PER_S, ROWS_PER_S)

        pltpu.sync_copy(zeros_hbm, acc.at[zrow])
        if split_edges:
            rows = pl.ds(cid * js, js)
            pltpu.sync_copy(edge_hbm.at[0].at[sid].at[rows], src_v)
            pltpu.sync_copy(edge_hbm.at[1].at[sid].at[rows], dst_v)
        else:
            pltpu.sync_copy(edge_hbm.at[0].at[sid], src_v)
            pltpu.sync_copy(edge_hbm.at[1].at[sid], dst_v)
        plsc.subcore_barrier()

        ngroups = js // nbuf
        tail = js - ngroups * nbuf

        def run(h_hbm):
            def group(g, carry):
                base = g * nbuf
                for b in range(nbuf):
                    @pl.when(g > 0)
                    def _():
                        pltpu.make_async_copy(
                            rows_v.at[b], acc.at[dst_v.at[base + b]],
                            ssem.at[b]).wait()
                    pltpu.async_copy(h_hbm.at[src_v.at[base + b]],
                                     rows_v.at[b], gsem.at[b])
                for b in range(nbuf):
                    pltpu.make_async_copy(h_hbm.at[src_v.at[base + b]],
                                          rows_v.at[b], gsem.at[b]).wait()
                    pltpu.async_copy(rows_v.at[b], acc.at[dst_v.at[base + b]],
                                     ssem.at[b], add=True)
                return carry

            lax.fori_loop(0, ngroups, group, 0)
            for u in range(tail):
                jt = ngroups * nbuf + u
                pltpu.make_async_copy(rows_v.at[u], acc.at[dst_v.at[jt]],
                                      ssem.at[u]).wait()
                pltpu.async_copy(h_hbm.at[src_v.at[jt]],
                                 rows_v.at[u], gsem.at[u]).wait()
                pltpu.async_copy(rows_v.at[u], acc.at[dst_v.at[jt]],
                                 ssem.at[u], add=True)
            for b in range(nbuf):
                pltpu.make_async_copy(rows_v.at[b], acc.at[dst_v.at[b]],
                                      ssem.at[b]).wait()

        if split_edges:
            run(h_hbms[0])
        else:
            @pl.when(cid == 0)
            def _():
                run(h_hbms[0])

            @pl.when(cid == 1)
            def _():
                run(h_hbms[1])

        plsc.subcore_barrier()
        pltpu.sync_copy(acc.at[zrow], out_hbm.at[cid].at[zrow])

    return agg


_agg64 = _make_agg(IN_F // 2, 8, split_edges=False)
_agg32 = _make_agg(OUT_F, 10, split_edges=True)


_R = 2000


def _pre_body(x_ref, deg_ref, h0_ref, h1_ref, ns_ref, nd_ref):
    ns = lax.rsqrt(jnp.maximum(deg_ref[0, :, :1], 1.0))
    nd = lax.rsqrt(jnp.maximum(deg_ref[1, :, :1], 1.0))
    h = x_ref[...] * ns
    h0_ref[...] = h[:, :IN_F // 2]
    h1_ref[...] = h[:, IN_F // 2:]
    ns_ref[...] = ns
    nd_ref[...] = nd


_pre = pl.pallas_call(
    _pre_body,
    grid=(N_NODES // _R,),
    in_specs=[
        pl.BlockSpec((_R, IN_F), lambda i: (i, 0)),
        pl.BlockSpec((2, _R, DEG_W), lambda i: (0, i, 0)),
    ],
    out_specs=[
        pl.BlockSpec((_R, IN_F // 2), lambda i: (i, 0)),
        pl.BlockSpec((_R, IN_F // 2), lambda i: (i, 0)),
        pl.BlockSpec((_R, 1), lambda i: (i, 0)),
        pl.BlockSpec((_R, 1), lambda i: (i, 0)),
    ],
    out_shape=[
        jax.ShapeDtypeStruct((N_NODES, IN_F // 2), jnp.float32),
        jax.ShapeDtypeStruct((N_NODES, IN_F // 2), jnp.float32),
        jax.ShapeDtypeStruct((N_NODES, 1), jnp.float32),
        jax.ShapeDtypeStruct((N_NODES, 1), jnp.float32),
    ],
)


def _mid_body(p_ref, nd_ref, ns_ref, w1_ref, b1_ref, w2_ref, t_ref):
    a0 = p_ref[0] * nd_ref[...]
    a1 = p_ref[1] * nd_ref[...]
    z = (jnp.dot(a0, w1_ref[:IN_F // 2], preferred_element_type=jnp.float32)
         + jnp.dot(a1, w1_ref[IN_F // 2:], preferred_element_type=jnp.float32)
         + b1_ref[...])
    r = jnp.maximum(z, 0.0)
    t_ref[...] = jnp.dot(r, w2_ref[...],
                         preferred_element_type=jnp.float32) * ns_ref[...]


_mid = pl.pallas_call(
    _mid_body,
    grid=(N_NODES // _R,),
    in_specs=[
        pl.BlockSpec((2, _R, IN_F // 2), lambda i: (0, i, 0)),
        pl.BlockSpec((_R, 1), lambda i: (i, 0)),
        pl.BlockSpec((_R, 1), lambda i: (i, 0)),
        pl.BlockSpec((IN_F, HID), lambda i: (0, 0)),
        pl.BlockSpec((1, HID), lambda i: (0, 0)),
        pl.BlockSpec((HID, OUT_F), lambda i: (0, 0)),
    ],
    out_specs=pl.BlockSpec((_R, OUT_F), lambda i: (i, 0)),
    out_shape=jax.ShapeDtypeStruct((N_NODES, OUT_F), jnp.float32),
)


def _out_body(q_ref, nd_ref, b2_ref, o_ref):
    o_ref[...] = (q_ref[0] + q_ref[1]) * nd_ref[...] + b2_ref[...]


_out = pl.pallas_call(
    _out_body,
    grid=(N_NODES // _R,),
    in_specs=[
        pl.BlockSpec((2, _R, OUT_F), lambda i: (0, i, 0)),
        pl.BlockSpec((_R, 1), lambda i: (i, 0)),
        pl.BlockSpec((1, OUT_F), lambda i: (0, 0)),
    ],
    out_specs=pl.BlockSpec((_R, OUT_F), lambda i: (i, 0)),
    out_shape=jax.ShapeDtypeStruct((N_NODES, OUT_F), jnp.float32),
)


def kernel(features, edge_index, W1, b1, W2, b2):
    edge4d = edge_index.astype(jnp.int32).reshape(2, N_SUB, JS, K)
    ones8 = jnp.ones((K, DEG_W), jnp.float32)
    zeros8 = jnp.zeros((ROWS_PER_S, DEG_W), jnp.float32)
    deg = _deg_kernel(edge4d, ones8, zeros8)
    h0, h1, ns, nd = _pre(features, deg)
    zeros64 = jnp.zeros((ROWS_PER_S, IN_F // 2), jnp.float32)
    p = _agg64(h0, h1, edge4d, zeros64)
    t = _mid(p, nd, ns, W1, b1.reshape(1, HID), W2)
    zeros32 = jnp.zeros((ROWS_PER_S, OUT_F), jnp.float32)
    q = _agg32(t, edge4d, zeros32)
    return _out(q, nd, b2.reshape(1, OUT_F))
